# baseline (speedup 1.0000x reference)
import jax
import jax.numpy as jnp
from jax.experimental import pallas as pl
from jax.experimental.pallas import tpu as pltpu

B = 128
D = 768
F = 65536
K = 64
BF = 2048
NB = F // BF


def _proj_body(x_ref, w_ref, out_ref):
    x = x_ref[...]
    w = w_ref[...]
    out_ref[...] = jax.lax.dot_general(
        x, w, (((1,), (1,)), ((), ())), preferred_element_type=jnp.float32
    )


def _project(embed0, W_enc):
    return pl.pallas_call(
        _proj_body,
        grid=(NB,),
        in_specs=[
            pl.BlockSpec((B, D), lambda i: (0, 0)),
            pl.BlockSpec((BF, D), lambda i: (i, 0)),
        ],
        out_specs=pl.BlockSpec((B, BF), lambda i: (0, i)),
        out_shape=jax.ShapeDtypeStruct((B, F), jnp.float32),
    )(embed0, W_enc)


def kernel(embed, W_enc, W_dec, bias):
    embed0 = embed - bias
    project = _project(embed0, W_enc)
    weights, feats = jax.lax.top_k(project, K)
    vecs = jnp.take(W_dec, feats, axis=0)
    recon = jnp.einsum("btd,bt->bd", vecs, weights) + bias
    norm = jnp.maximum(jnp.linalg.norm(recon, axis=-1, keepdims=True), 1e-12)
    return recon / norm


# speedup vs baseline: 5.0992x; 5.0992x over previous
import jax
import jax.numpy as jnp
from jax import lax
from jax.experimental import pallas as pl
from jax.experimental.pallas import tpu as pltpu

B = 128
D = 768
F = 65536
K = 64
BF = 1024
NB = F // BF
CW = 2048
NCH = F // CW
NBLK = 512
NSTR = F // NBLK
NLVL = 6

_I32_MIN = -2147483648
_MASK31 = 0x7FFFFFFF
_NEG_INF = float("-inf")


def _mono_key(x):
    b = lax.bitcast_convert_type(x, jnp.int32)
    return jnp.where(b < 0, b ^ _MASK31, b)


def _unmono(k):
    b = jnp.where(k < 0, k ^ _MASK31, k)
    return lax.bitcast_convert_type(b, jnp.float32)


def _body(embed_ref, bias_ref, wenc_ref, wdec_ref, out_ref,
          sc_ref, x_ref, recon_ref, t_ref, m_ref, mt_ref):
    i = pl.program_id(0)

    @pl.when(i == 0)
    def _init():
        x_ref[...] = embed_ref[...] - bias_ref[...]
        recon_ref[...] = jnp.zeros((B, D), jnp.float32)

    @pl.when(i < NB)
    def _encode():
        s = lax.dot_general(x_ref[...], wenc_ref[...],
                            (((1,), (1,)), ((), ())),
                            preferred_element_type=jnp.float32)
        sc_ref[:, pl.ds(pl.multiple_of(i * BF, BF), BF)] = s

    @pl.when(i == NB - 1)
    def _select():
        def count_ge(cand):
            def chunk(c, acc):
                s = sc_ref[:, pl.ds(pl.multiple_of(c * CW, CW), CW)]
                return acc + (_mono_key(s) >= cand).astype(jnp.int32)
            acc = lax.fori_loop(0, NCH, chunk, jnp.zeros((B, CW), jnp.int32))
            return jnp.sum(acc, axis=1, keepdims=True)

        prev = None
        for l in range(NLVL):
            ml = jnp.full((B, NBLK), _NEG_INF, jnp.float32)
            for m in range(NSTR):
                sl = sc_ref[:, m * NBLK:(m + 1) * NBLK]
                if prev is not None:
                    sl = jnp.where(sl < prev, sl, _NEG_INF)
                ml = jnp.maximum(ml, sl)
            mt_ref[:, l * NBLK:(l + 1) * NBLK] = ml
            prev = ml

        mkeys = _mono_key(lax.transpose(mt_ref[...], (1, 0)))

        def mcount(cand):
            return jnp.sum((mkeys >= cand).astype(jnp.int32),
                           axis=0, keepdims=True)

        def ms_iter(j, t):
            cand = t + (jnp.int32(1) << (30 - j).astype(jnp.int32))
            return jnp.where(mcount(cand) >= K, cand, t)

        tm0 = jnp.where(mcount(jnp.zeros((1, B), jnp.int32)) >= K,
                        0, _I32_MIN).astype(jnp.int32)
        tm = lax.fori_loop(0, 31, ms_iter, tm0)

        def to_col(row):
            return lax.transpose(jnp.broadcast_to(row, (B, B)), (1, 0))[:, 0:1]

        tm_col = to_col(tm)
        mc_col = to_col(mcount(tm))
        ok = jnp.sum(jnp.where(count_ge(tm_col) == mc_col, 0, 1)) == 0
        t_ref[...] = jnp.broadcast_to(tm_col, (B, 128))

        @pl.when(jnp.logical_not(ok))
        def _full_search():
            def bs_iter(j, t):
                cand = t + (jnp.int32(1) << (30 - j).astype(jnp.int32))
                return jnp.where(count_ge(cand) >= K, cand, t)

            t0 = jnp.where(count_ge(jnp.zeros((B, 1), jnp.int32)) >= K,
                           0, _I32_MIN).astype(jnp.int32)
            tf = lax.fori_loop(0, 31, bs_iter, t0)
            t_ref[...] = jnp.broadcast_to(tf, (B, 128))

        t = t_ref[:, 0:1]
        cnt_gt = count_ge(t + 1)
        need = K - cnt_gt
        cnt_ge = count_ge(t)
        any_tie = jnp.sum(jnp.where(cnt_ge != K, 1, 0)) > 0
        m_ref[...] = jnp.full((B, 128), F, jnp.int32)

        @pl.when(any_tie)
        def _break_ties():
            tf32 = _unmono(t)

            def count_eq_lt(cand):
                def chunk(c, acc):
                    s = sc_ref[:, pl.ds(pl.multiple_of(c * CW, CW), CW)]
                    idx = (lax.broadcasted_iota(jnp.int32, (B, CW), 1)
                           + c * CW)
                    hit = (s == tf32) & (idx < cand)
                    return acc + hit.astype(jnp.int32)
                acc = lax.fori_loop(0, NCH, chunk,
                                    jnp.zeros((B, CW), jnp.int32))
                return jnp.sum(acc, axis=1, keepdims=True)

            def m_iter(j, m):
                cand = m + (jnp.int32(1) << (16 - j).astype(jnp.int32))
                c = count_eq_lt(cand)
                return jnp.where(c < need, cand, m)

            m = lax.fori_loop(0, 17, m_iter, jnp.zeros((B, 1), jnp.int32))
            m_ref[...] = jnp.broadcast_to(m + 1, (B, 128))

    @pl.when(i >= NB)
    def _decode():
        blk = i - NB
        s = sc_ref[:, pl.ds(pl.multiple_of(blk * BF, BF), BF)]
        tf32 = _unmono(t_ref[:, 0:1])
        m = m_ref[:, 0:1]
        idx = lax.broadcasted_iota(jnp.int32, (B, BF), 1) + blk * BF
        sel = (s > tf32) | ((s == tf32) & (idx < m))
        vals = jnp.where(sel, s, 0.0).astype(jnp.bfloat16)
        recon_ref[...] += lax.dot_general(
            vals, wdec_ref[...], (((1,), (0,)), ((), ())),
            preferred_element_type=jnp.float32)

    @pl.when(i == 2 * NB - 1)
    def _finish():
        recon = recon_ref[...] + bias_ref[...]
        s = jnp.sum(recon * recon, axis=1, keepdims=True)
        norm = jnp.maximum(jnp.sqrt(s), 1e-12)
        out_ref[...] = recon / norm


def kernel(embed, W_enc, W_dec, bias):
    bias2 = bias.reshape(1, D)
    wdec_bf16 = W_dec.astype(jnp.bfloat16)
    return pl.pallas_call(
        _body,
        grid=(2 * NB,),
        in_specs=[
            pl.BlockSpec((B, D), lambda i: (0, 0)),
            pl.BlockSpec((1, D), lambda i: (0, 0)),
            pl.BlockSpec((BF, D), lambda i: (jnp.minimum(i, NB - 1), 0)),
            pl.BlockSpec((BF, D), lambda i: (jnp.maximum(i - NB, 0), 0)),
        ],
        out_specs=pl.BlockSpec((B, D), lambda i: (0, 0)),
        out_shape=jax.ShapeDtypeStruct((B, D), jnp.float32),
        scratch_shapes=[
            pltpu.VMEM((B, F), jnp.float32),
            pltpu.VMEM((B, D), jnp.float32),
            pltpu.VMEM((B, D), jnp.float32),
            pltpu.VMEM((B, 128), jnp.int32),
            pltpu.VMEM((B, 128), jnp.int32),
            pltpu.VMEM((B, NLVL * NBLK), jnp.float32),
        ],
    )(embed, bias2, W_enc, wdec_bf16)


# speedup vs baseline: 5.7112x; 1.1200x over previous
import jax
import jax.numpy as jnp
from jax import lax
from jax.experimental import pallas as pl
from jax.experimental.pallas import tpu as pltpu

B = 128
D = 768
F = 65536
K = 64
BF = 1024
NB = F // BF
CW = 2048
NCH = F // CW
NBLK = 512
NSTR = F // NBLK
NLVL = 6

_I32_MIN = -2147483648
_MASK31 = 0x7FFFFFFF
_NEG_INF = float("-inf")


def _mono_key(x):
    b = lax.bitcast_convert_type(x, jnp.int32)
    return jnp.where(b < 0, b ^ _MASK31, b)


def _unmono(k):
    b = jnp.where(k < 0, k ^ _MASK31, k)
    return lax.bitcast_convert_type(b, jnp.float32)


def _body(embed_ref, bias_ref, wenc_ref, wdec_ref, out_ref,
          sc_ref, x_ref, recon_ref, t_ref, m_ref, mt_ref):
    i = pl.program_id(0)

    @pl.when(i == 0)
    def _init():
        x_ref[...] = embed_ref[...] - bias_ref[...]
        recon_ref[...] = jnp.zeros((B, D), jnp.float32)
        mt_ref[...] = jnp.full((B, NLVL * NBLK), _NEG_INF, jnp.float32)

    @pl.when(i < NB)
    def _encode():
        s = lax.dot_general(x_ref[...], wenc_ref[...],
                            (((1,), (1,)), ((), ())),
                            preferred_element_type=jnp.float32)
        sc_ref[:, pl.ds(pl.multiple_of(i * BF, BF), BF)] = s
        for sub in range(BF // NBLK):
            v = s[:, sub * NBLK:(sub + 1) * NBLK]
            for l in range(NLVL):
                cur = mt_ref[:, l * NBLK:(l + 1) * NBLK]
                hi = jnp.maximum(cur, v)
                v = jnp.minimum(cur, v)
                mt_ref[:, l * NBLK:(l + 1) * NBLK] = hi

    @pl.when(i == NB - 1)
    def _select():
        def count_ge(cand):
            def chunk(c, acc):
                s = sc_ref[:, pl.ds(pl.multiple_of(c * CW, CW), CW)]
                return acc + (_mono_key(s) >= cand).astype(jnp.int32)
            acc = lax.fori_loop(0, NCH, chunk, jnp.zeros((B, CW), jnp.int32))
            return jnp.sum(acc, axis=1, keepdims=True)

        def count_ge2(cand):
            def chunk(c, accs):
                a1, a2 = accs
                s = sc_ref[:, pl.ds(pl.multiple_of(c * CW, CW), CW)]
                k = _mono_key(s)
                return (a1 + (k >= cand).astype(jnp.int32),
                        a2 + (k >= cand + 1).astype(jnp.int32))
            z = jnp.zeros((B, CW), jnp.int32)
            a1, a2 = lax.fori_loop(0, NCH, chunk, (z, z))
            return (jnp.sum(a1, axis=1, keepdims=True),
                    jnp.sum(a2, axis=1, keepdims=True))

        mkeys = _mono_key(lax.transpose(mt_ref[...], (1, 0)))

        def mcount(cand):
            return jnp.sum((mkeys >= cand).astype(jnp.int32),
                           axis=0, keepdims=True)

        def ms_iter(j, t):
            cand = t + (jnp.int32(1) << (30 - j).astype(jnp.int32))
            return jnp.where(mcount(cand) >= K, cand, t)

        tm0 = jnp.where(mcount(jnp.zeros((1, B), jnp.int32)) >= K,
                        0, _I32_MIN).astype(jnp.int32)
        tm = lax.fori_loop(0, 31, ms_iter, tm0)

        def to_col(row):
            return lax.transpose(jnp.broadcast_to(row, (B, B)), (1, 0))[:, 0:1]

        tm_col = to_col(tm)
        mc_col = to_col(mcount(tm))
        cge_tm, cgt_tm = count_ge2(tm_col)
        ok = jnp.sum(jnp.where(cge_tm == mc_col, 0, 1)) == 0

        def finish_select(t_col, cnt_ge, cnt_gt):
            t_ref[...] = jnp.broadcast_to(t_col, (B, 128))
            need = K - cnt_gt
            any_tie = jnp.sum(jnp.where(cnt_ge != K, 1, 0)) > 0
            m_ref[...] = jnp.full((B, 128), F, jnp.int32)

            @pl.when(any_tie)
            def _break_ties():
                tf32 = _unmono(t_col)

                def count_eq_lt(cand):
                    def chunk(c, acc):
                        s = sc_ref[:, pl.ds(pl.multiple_of(c * CW, CW), CW)]
                        idx = (lax.broadcasted_iota(jnp.int32, (B, CW), 1)
                               + c * CW)
                        hit = (s == tf32) & (idx < cand)
                        return acc + hit.astype(jnp.int32)
                    acc = lax.fori_loop(0, NCH, chunk,
                                        jnp.zeros((B, CW), jnp.int32))
                    return jnp.sum(acc, axis=1, keepdims=True)

                def m_iter(j, m):
                    cand = m + (jnp.int32(1) << (16 - j).astype(jnp.int32))
                    c = count_eq_lt(cand)
                    return jnp.where(c < need, cand, m)

                m = lax.fori_loop(0, 17, m_iter,
                                  jnp.zeros((B, 1), jnp.int32))
                m_ref[...] = jnp.broadcast_to(m + 1, (B, 128))

        @pl.when(ok)
        def _fast():
            finish_select(tm_col, cge_tm, cgt_tm)

        @pl.when(jnp.logical_not(ok))
        def _full_search():
            def bs_iter(j, t):
                cand = t + (jnp.int32(1) << (30 - j).astype(jnp.int32))
                return jnp.where(count_ge(cand) >= K, cand, t)

            t0 = jnp.where(count_ge(jnp.zeros((B, 1), jnp.int32)) >= K,
                           0, _I32_MIN).astype(jnp.int32)
            tf = lax.fori_loop(0, 31, bs_iter, t0)
            cge_f, cgt_f = count_ge2(tf)
            finish_select(tf, cge_f, cgt_f)

    @pl.when(i >= NB)
    def _decode():
        blk = i - NB
        s = sc_ref[:, pl.ds(pl.multiple_of(blk * BF, BF), BF)]
        tf32 = _unmono(t_ref[:, 0:1])
        m = m_ref[:, 0:1]
        idx = lax.broadcasted_iota(jnp.int32, (B, BF), 1) + blk * BF
        sel = (s > tf32) | ((s == tf32) & (idx < m))
        vals = jnp.where(sel, s, 0.0).astype(jnp.bfloat16)
        recon_ref[...] += lax.dot_general(
            vals, wdec_ref[...], (((1,), (0,)), ((), ())),
            preferred_element_type=jnp.float32)

    @pl.when(i == 2 * NB - 1)
    def _finish():
        recon = recon_ref[...] + bias_ref[...]
        s = jnp.sum(recon * recon, axis=1, keepdims=True)
        norm = jnp.maximum(jnp.sqrt(s), 1e-12)
        out_ref[...] = recon / norm


def kernel(embed, W_enc, W_dec, bias):
    bias2 = bias.reshape(1, D)
    wdec_bf16 = W_dec.astype(jnp.bfloat16)
    return pl.pallas_call(
        _body,
        grid=(2 * NB,),
        in_specs=[
            pl.BlockSpec((B, D), lambda i: (0, 0)),
            pl.BlockSpec((1, D), lambda i: (0, 0)),
            pl.BlockSpec((BF, D), lambda i: (jnp.minimum(i, NB - 1), 0)),
            pl.BlockSpec((BF, D), lambda i: (jnp.maximum(i - NB, 0), 0)),
        ],
        out_specs=pl.BlockSpec((B, D), lambda i: (0, 0)),
        out_shape=jax.ShapeDtypeStruct((B, D), jnp.float32),
        scratch_shapes=[
            pltpu.VMEM((B, F), jnp.float32),
            pltpu.VMEM((B, D), jnp.float32),
            pltpu.VMEM((B, D), jnp.float32),
            pltpu.VMEM((B, 128), jnp.int32),
            pltpu.VMEM((B, 128), jnp.int32),
            pltpu.VMEM((B, NLVL * NBLK), jnp.float32),
        ],
    )(embed, bias2, W_enc, wdec_bf16)


# speedup vs baseline: 5.8448x; 1.0234x over previous
import jax
import jax.numpy as jnp
from jax import lax
from jax.experimental import pallas as pl
from jax.experimental.pallas import tpu as pltpu

B = 128
D = 768
F = 65536
K = 64
BF = 1024
NB = F // BF
CW = 2048
NCH = F // CW
NBLK = 512
NSTR = F // NBLK
NLVL = 6

_I32_MIN = -2147483648
_MASK31 = 0x7FFFFFFF
_NEG_INF = float("-inf")


def _mono_key(x):
    b = lax.bitcast_convert_type(x, jnp.int32)
    return jnp.where(b < 0, b ^ _MASK31, b)


def _unmono(k):
    b = jnp.where(k < 0, k ^ _MASK31, k)
    return lax.bitcast_convert_type(b, jnp.float32)


def _body(embed_ref, bias_ref, wenc_ref, wdec_ref, out_ref,
          sc_ref, x_ref, recon_ref, t_ref, m_ref, mt_ref):
    i = pl.program_id(0)

    @pl.when(i == 0)
    def _init():
        x_ref[...] = embed_ref[...] - bias_ref[...]
        recon_ref[...] = jnp.zeros((B, D), jnp.float32)
        mt_ref[...] = jnp.full((B, NLVL * NBLK), _NEG_INF, jnp.float32)

    @pl.when(i < NB)
    def _encode():
        s = lax.dot_general(x_ref[...], wenc_ref[...],
                            (((1,), (1,)), ((), ())),
                            preferred_element_type=jnp.float32)
        sc_ref[:, pl.ds(pl.multiple_of(i * BF, BF), BF)] = s
        for sub in range(BF // NBLK):
            v = s[:, sub * NBLK:(sub + 1) * NBLK]
            for l in range(NLVL):
                cur = mt_ref[:, l * NBLK:(l + 1) * NBLK]
                hi = jnp.maximum(cur, v)
                v = jnp.minimum(cur, v)
                mt_ref[:, l * NBLK:(l + 1) * NBLK] = hi

    @pl.when(i == NB - 1)
    def _select():
        def count_ge(cand):
            def chunk(c, acc):
                s = sc_ref[:, pl.ds(pl.multiple_of(c * CW, CW), CW)]
                hit = (_mono_key(s) >= cand).astype(jnp.int32)
                return acc + jnp.sum(hit, axis=1, keepdims=True)
            return lax.fori_loop(0, NCH, chunk, jnp.zeros((B, 1), jnp.int32))

        def count_ge2(cand):
            def chunk(c, accs):
                a1, a2 = accs
                s = sc_ref[:, pl.ds(pl.multiple_of(c * CW, CW), CW)]
                k = _mono_key(s)
                h1 = (k >= cand).astype(jnp.int32)
                h2 = (k >= cand + 1).astype(jnp.int32)
                return (a1 + jnp.sum(h1, axis=1, keepdims=True),
                        a2 + jnp.sum(h2, axis=1, keepdims=True))
            z = jnp.zeros((B, 1), jnp.int32)
            return lax.fori_loop(0, NCH, chunk, (z, z))

        mkeys = _mono_key(lax.transpose(mt_ref[...], (1, 0)))

        def mcount(cand):
            return jnp.sum((mkeys >= cand).astype(jnp.int32),
                           axis=0, keepdims=True)

        def ms_iter(j, t):
            cand = t + (jnp.int32(1) << (30 - j).astype(jnp.int32))
            return jnp.where(mcount(cand) >= K, cand, t)

        tm0 = jnp.where(mcount(jnp.zeros((1, B), jnp.int32)) >= K,
                        0, _I32_MIN).astype(jnp.int32)
        tm = lax.fori_loop(0, 31, ms_iter, tm0)

        def to_col(row):
            return lax.transpose(jnp.broadcast_to(row, (B, B)), (1, 0))[:, 0:1]

        tm_col = to_col(tm)
        mc_col = to_col(mcount(tm))
        cge_tm, cgt_tm = count_ge2(tm_col)
        ok = jnp.sum(jnp.where(cge_tm == mc_col, 0, 1)) == 0

        def finish_select(t_col, cnt_ge, cnt_gt):
            t_ref[...] = jnp.broadcast_to(t_col, (B, 128))
            need = K - cnt_gt
            any_tie = jnp.sum(jnp.where(cnt_ge != K, 1, 0)) > 0
            m_ref[...] = jnp.full((B, 128), F, jnp.int32)

            @pl.when(any_tie)
            def _break_ties():
                tf32 = _unmono(t_col)

                def count_eq_lt(cand):
                    def chunk(c, acc):
                        s = sc_ref[:, pl.ds(pl.multiple_of(c * CW, CW), CW)]
                        idx = (lax.broadcasted_iota(jnp.int32, (B, CW), 1)
                               + c * CW)
                        hit = ((s == tf32) & (idx < cand)).astype(jnp.int32)
                        return acc + jnp.sum(hit, axis=1, keepdims=True)
                    return lax.fori_loop(0, NCH, chunk,
                                         jnp.zeros((B, 1), jnp.int32))

                def m_iter(j, m):
                    cand = m + (jnp.int32(1) << (16 - j).astype(jnp.int32))
                    c = count_eq_lt(cand)
                    return jnp.where(c < need, cand, m)

                m = lax.fori_loop(0, 17, m_iter,
                                  jnp.zeros((B, 1), jnp.int32))
                m_ref[...] = jnp.broadcast_to(m + 1, (B, 128))

        @pl.when(ok)
        def _fast():
            finish_select(tm_col, cge_tm, cgt_tm)

        @pl.when(jnp.logical_not(ok))
        def _full_search():
            def bs_iter(j, t):
                cand = t + (jnp.int32(1) << (30 - j).astype(jnp.int32))
                return jnp.where(count_ge(cand) >= K, cand, t)

            t0 = jnp.where(count_ge(jnp.zeros((B, 1), jnp.int32)) >= K,
                           0, _I32_MIN).astype(jnp.int32)
            tf = lax.fori_loop(0, 31, bs_iter, t0)
            cge_f, cgt_f = count_ge2(tf)
            finish_select(tf, cge_f, cgt_f)

    @pl.when(i >= NB)
    def _decode():
        blk = i - NB
        s = sc_ref[:, pl.ds(pl.multiple_of(blk * BF, BF), BF)]
        tf32 = _unmono(t_ref[:, 0:1])
        m = m_ref[:, 0:1]
        idx = lax.broadcasted_iota(jnp.int32, (B, BF), 1) + blk * BF
        sel = (s > tf32) | ((s == tf32) & (idx < m))
        vals = jnp.where(sel, s, 0.0).astype(jnp.bfloat16)
        recon_ref[...] += lax.dot_general(
            vals, wdec_ref[...], (((1,), (0,)), ((), ())),
            preferred_element_type=jnp.float32)

    @pl.when(i == 2 * NB - 1)
    def _finish():
        recon = recon_ref[...] + bias_ref[...]
        s = jnp.sum(recon * recon, axis=1, keepdims=True)
        norm = jnp.maximum(jnp.sqrt(s), 1e-12)
        out_ref[...] = recon / norm


def kernel(embed, W_enc, W_dec, bias):
    bias2 = bias.reshape(1, D)
    wdec_bf16 = W_dec.astype(jnp.bfloat16)
    return pl.pallas_call(
        _body,
        grid=(2 * NB,),
        in_specs=[
            pl.BlockSpec((B, D), lambda i: (0, 0)),
            pl.BlockSpec((1, D), lambda i: (0, 0)),
            pl.BlockSpec((BF, D), lambda i: (jnp.minimum(i, NB - 1), 0)),
            pl.BlockSpec((BF, D), lambda i: (jnp.maximum(i - NB, 0), 0)),
        ],
        out_specs=pl.BlockSpec((B, D), lambda i: (0, 0)),
        out_shape=jax.ShapeDtypeStruct((B, D), jnp.float32),
        scratch_shapes=[
            pltpu.VMEM((B, F), jnp.float32),
            pltpu.VMEM((B, D), jnp.float32),
            pltpu.VMEM((B, D), jnp.float32),
            pltpu.VMEM((B, 128), jnp.int32),
            pltpu.VMEM((B, 128), jnp.int32),
            pltpu.VMEM((B, NLVL * NBLK), jnp.float32),
        ],
    )(embed, bias2, W_enc, wdec_bf16)


# speedup vs baseline: 5.9137x; 1.0118x over previous
import jax
import jax.numpy as jnp
from jax import lax
from jax.experimental import pallas as pl
from jax.experimental.pallas import tpu as pltpu

B = 128
D = 768
F = 65536
K = 64
BF = 1024
NB = F // BF
CW = 2048
NCH = F // CW
NBLK = 512
NSTR = F // NBLK
NLVL = 6

_I32_MIN = -2147483648
_MASK31 = 0x7FFFFFFF
_NEG_INF = float("-inf")


def _mono_key(x):
    b = lax.bitcast_convert_type(x, jnp.int32)
    return jnp.where(b < 0, b ^ _MASK31, b)


def _unmono(k):
    b = jnp.where(k < 0, k ^ _MASK31, k)
    return lax.bitcast_convert_type(b, jnp.float32)


def _body(embed_ref, bias_ref, wenc_ref, wdec_ref, out_ref,
          sc_ref, x_ref, recon_ref, t_ref, m_ref, mt_ref):
    i = pl.program_id(0)

    @pl.when(i == 0)
    def _init():
        x_ref[...] = embed_ref[...] - bias_ref[...]
        recon_ref[...] = jnp.zeros((B, D), jnp.float32)
        mt_ref[...] = jnp.full((B, NLVL * NBLK), _NEG_INF, jnp.float32)

    @pl.when(i < NB)
    def _encode():
        s = lax.dot_general(x_ref[...], wenc_ref[...],
                            (((1,), (1,)), ((), ())),
                            preferred_element_type=jnp.float32)
        sc_ref[:, pl.ds(pl.multiple_of(i * BF, BF), BF)] = s
        for sub in range(BF // NBLK):
            v = s[:, sub * NBLK:(sub + 1) * NBLK]
            for l in range(NLVL):
                cur = mt_ref[:, l * NBLK:(l + 1) * NBLK]
                hi = jnp.maximum(cur, v)
                v = jnp.minimum(cur, v)
                mt_ref[:, l * NBLK:(l + 1) * NBLK] = hi

    @pl.when(i == NB - 1)
    def _select():
        def count_ge(cand):
            def chunk(c, acc):
                s = sc_ref[:, pl.ds(pl.multiple_of(c * CW, CW), CW)]
                hit = (_mono_key(s) >= cand).astype(jnp.int32)
                return acc + jnp.sum(hit, axis=1, keepdims=True)
            return lax.fori_loop(0, NCH, chunk, jnp.zeros((B, 1), jnp.int32))

        def count_ge2(cand):
            def chunk(c, accs):
                a1, a2 = accs
                s = sc_ref[:, pl.ds(pl.multiple_of(c * CW, CW), CW)]
                k = _mono_key(s)
                h1 = (k >= cand).astype(jnp.int32)
                h2 = (k >= cand + 1).astype(jnp.int32)
                return (a1 + jnp.sum(h1, axis=1, keepdims=True),
                        a2 + jnp.sum(h2, axis=1, keepdims=True))
            z = jnp.zeros((B, 1), jnp.int32)
            return lax.fori_loop(0, NCH, chunk, (z, z))

        mkeys = _mono_key(lax.transpose(mt_ref[...], (1, 0)))

        def mcount(cand):
            return jnp.sum((mkeys >= cand).astype(jnp.int32),
                           axis=0, keepdims=True)

        def ms_iter(j, t):
            cand = t + (jnp.int32(1) << (30 - j).astype(jnp.int32))
            return jnp.where(mcount(cand) >= K, cand, t)

        tm0 = jnp.where(mcount(jnp.zeros((1, B), jnp.int32)) >= K,
                        0, _I32_MIN).astype(jnp.int32)
        tm = lax.fori_loop(0, 31, ms_iter, tm0)

        def to_col(row):
            return lax.transpose(jnp.broadcast_to(row, (B, B)), (1, 0))[:, 0:1]

        tm_col = to_col(tm)
        mc_col = to_col(mcount(tm))
        cge_tm, cgt_tm = count_ge2(tm_col)
        ok = jnp.sum(jnp.where(cge_tm == mc_col, 0, 1)) == 0
        any_tie = jnp.sum(jnp.where(cge_tm != K, 1, 0)) > 0

        t_ref[...] = jnp.broadcast_to(tm_col, (B, 128))
        m_ref[...] = jnp.full((B, 128), F, jnp.int32)

        @pl.when(jnp.logical_not(ok) | any_tie)
        def _slow_exact():
            def bs_iter(j, t):
                cand = t + (jnp.int32(1) << (30 - j).astype(jnp.int32))
                return jnp.where(count_ge(cand) >= K, cand, t)

            t0 = jnp.where(count_ge(jnp.zeros((B, 1), jnp.int32)) >= K,
                           0, _I32_MIN).astype(jnp.int32)
            tf = lax.fori_loop(0, 31, bs_iter, t0)
            _, cgt_f = count_ge2(tf)
            t_ref[...] = jnp.broadcast_to(tf, (B, 128))
            need = K - cgt_f
            tf32 = _unmono(tf)

            def count_eq_lt(cand):
                def chunk(c, acc):
                    s = sc_ref[:, pl.ds(pl.multiple_of(c * CW, CW), CW)]
                    idx = (lax.broadcasted_iota(jnp.int32, (B, CW), 1)
                           + c * CW)
                    hit = ((s == tf32) & (idx < cand)).astype(jnp.int32)
                    return acc + jnp.sum(hit, axis=1, keepdims=True)
                return lax.fori_loop(0, NCH, chunk,
                                     jnp.zeros((B, 1), jnp.int32))

            def m_iter(j, m):
                cand = m + (jnp.int32(1) << (16 - j).astype(jnp.int32))
                c = count_eq_lt(cand)
                return jnp.where(c < need, cand, m)

            m = lax.fori_loop(0, 17, m_iter, jnp.zeros((B, 1), jnp.int32))
            m_ref[...] = jnp.broadcast_to(m + 1, (B, 128))

    @pl.when(i >= NB)
    def _decode():
        blk = i - NB
        s = sc_ref[:, pl.ds(pl.multiple_of(blk * BF, BF), BF)]
        tf32 = _unmono(t_ref[:, 0:1])
        m = m_ref[:, 0:1]
        idx = lax.broadcasted_iota(jnp.int32, (B, BF), 1) + blk * BF
        sel = (s > tf32) | ((s == tf32) & (idx < m))
        vals = jnp.where(sel, s, 0.0).astype(jnp.bfloat16)
        recon_ref[...] += lax.dot_general(
            vals, wdec_ref[...], (((1,), (0,)), ((), ())),
            preferred_element_type=jnp.float32)

    @pl.when(i == 2 * NB - 1)
    def _finish():
        recon = recon_ref[...] + bias_ref[...]
        s = jnp.sum(recon * recon, axis=1, keepdims=True)
        norm = jnp.maximum(jnp.sqrt(s), 1e-12)
        out_ref[...] = recon / norm


def kernel(embed, W_enc, W_dec, bias):
    bias2 = bias.reshape(1, D)
    wdec_bf16 = W_dec.astype(jnp.bfloat16)
    return pl.pallas_call(
        _body,
        grid=(2 * NB,),
        in_specs=[
            pl.BlockSpec((B, D), lambda i: (0, 0)),
            pl.BlockSpec((1, D), lambda i: (0, 0)),
            pl.BlockSpec((BF, D), lambda i: (jnp.minimum(i, NB - 1), 0)),
            pl.BlockSpec((BF, D), lambda i: (jnp.maximum(i - NB, 0), 0)),
        ],
        out_specs=pl.BlockSpec((B, D), lambda i: (0, 0)),
        out_shape=jax.ShapeDtypeStruct((B, D), jnp.float32),
        scratch_shapes=[
            pltpu.VMEM((B, F), jnp.float32),
            pltpu.VMEM((B, D), jnp.float32),
            pltpu.VMEM((B, D), jnp.float32),
            pltpu.VMEM((B, 128), jnp.int32),
            pltpu.VMEM((B, 128), jnp.int32),
            pltpu.VMEM((B, NLVL * NBLK), jnp.float32),
        ],
    )(embed, bias2, W_enc, wdec_bf16)


# speedup vs baseline: 6.3248x; 1.0695x over previous
import jax
import jax.numpy as jnp
from jax import lax
from jax.experimental import pallas as pl
from jax.experimental.pallas import tpu as pltpu

B = 128
D = 768
F = 65536
K = 64
BF = 1024
NB = F // BF
BFD = 2048
NBD = F // BFD
CW = 2048
NCH = F // CW
NBLK = 512
NSTR = F // NBLK
NLVL = 6

_I32_MIN = -2147483648
_MASK31 = 0x7FFFFFFF
_NEG_INF = float("-inf")


def _mono_key(x):
    b = lax.bitcast_convert_type(x, jnp.int32)
    return jnp.where(b < 0, b ^ _MASK31, b)


def _unmono(k):
    b = jnp.where(k < 0, k ^ _MASK31, k)
    return lax.bitcast_convert_type(b, jnp.float32)


def _body(embed_ref, bias_ref, wenc_ref, wdec_ref, out_ref,
          sc_ref, x_ref, recon_ref, t_ref, m_ref, mt_ref):
    i = pl.program_id(0)

    @pl.when(i == 0)
    def _init():
        x_ref[...] = embed_ref[...] - bias_ref[...]
        recon_ref[...] = jnp.zeros((B, D), jnp.float32)
        mt_ref[...] = jnp.full((B, NLVL * NBLK), _NEG_INF, jnp.float32)

    @pl.when(i < NB)
    def _encode():
        s = lax.dot_general(x_ref[...], wenc_ref[...],
                            (((1,), (1,)), ((), ())),
                            preferred_element_type=jnp.float32)
        sc_ref[:, pl.ds(pl.multiple_of(i * BF, BF), BF)] = s
        for sub in range(BF // NBLK):
            v = s[:, sub * NBLK:(sub + 1) * NBLK]
            for l in range(NLVL):
                cur = mt_ref[:, l * NBLK:(l + 1) * NBLK]
                hi = jnp.maximum(cur, v)
                v = jnp.minimum(cur, v)
                mt_ref[:, l * NBLK:(l + 1) * NBLK] = hi

    @pl.when(i == NB - 1)
    def _select():
        def count_ge(cand):
            def chunk(c, acc):
                s = sc_ref[:, pl.ds(pl.multiple_of(c * CW, CW), CW)]
                hit = (_mono_key(s) >= cand).astype(jnp.int32)
                return acc + jnp.sum(hit, axis=1, keepdims=True)
            return lax.fori_loop(0, NCH, chunk, jnp.zeros((B, 1), jnp.int32))

        def count_ge2(cand):
            def chunk(c, accs):
                a1, a2 = accs
                s = sc_ref[:, pl.ds(pl.multiple_of(c * CW, CW), CW)]
                k = _mono_key(s)
                h1 = (k >= cand).astype(jnp.int32)
                h2 = (k >= cand + 1).astype(jnp.int32)
                return (a1 + jnp.sum(h1, axis=1, keepdims=True),
                        a2 + jnp.sum(h2, axis=1, keepdims=True))
            z = jnp.zeros((B, 1), jnp.int32)
            return lax.fori_loop(0, NCH, chunk, (z, z))

        mkeys = _mono_key(lax.transpose(mt_ref[...], (1, 0)))

        def mcount(cand):
            return jnp.sum((mkeys >= cand).astype(jnp.int32),
                           axis=0, keepdims=True)

        def ms_iter(j, t):
            cand = t + (jnp.int32(1) << (30 - j).astype(jnp.int32))
            return jnp.where(mcount(cand) >= K, cand, t)

        tm0 = jnp.where(mcount(jnp.zeros((1, B), jnp.int32)) >= K,
                        0, _I32_MIN).astype(jnp.int32)
        tm = lax.fori_loop(0, 31, ms_iter, tm0)

        def to_col(row):
            return lax.transpose(jnp.broadcast_to(row, (B, B)), (1, 0))[:, 0:1]

        tm_col = to_col(tm)
        mc_col = to_col(mcount(tm))
        cge_tm, cgt_tm = count_ge2(tm_col)
        ok = jnp.sum(jnp.where(cge_tm == mc_col, 0, 1)) == 0
        any_tie = jnp.sum(jnp.where(cge_tm != K, 1, 0)) > 0

        t_ref[...] = jnp.broadcast_to(tm_col, (B, 128))
        m_ref[...] = jnp.full((B, 128), F, jnp.int32)

        @pl.when(jnp.logical_not(ok) | any_tie)
        def _slow_exact():
            def bs_iter(j, t):
                cand = t + (jnp.int32(1) << (30 - j).astype(jnp.int32))
                return jnp.where(count_ge(cand) >= K, cand, t)

            t0 = jnp.where(count_ge(jnp.zeros((B, 1), jnp.int32)) >= K,
                           0, _I32_MIN).astype(jnp.int32)
            tf = lax.fori_loop(0, 31, bs_iter, t0)
            _, cgt_f = count_ge2(tf)
            t_ref[...] = jnp.broadcast_to(tf, (B, 128))
            need = K - cgt_f
            tf32 = _unmono(tf)

            def count_eq_lt(cand):
                def chunk(c, acc):
                    s = sc_ref[:, pl.ds(pl.multiple_of(c * CW, CW), CW)]
                    idx = (lax.broadcasted_iota(jnp.int32, (B, CW), 1)
                           + c * CW)
                    hit = ((s == tf32) & (idx < cand)).astype(jnp.int32)
                    return acc + jnp.sum(hit, axis=1, keepdims=True)
                return lax.fori_loop(0, NCH, chunk,
                                     jnp.zeros((B, 1), jnp.int32))

            def m_iter(j, m):
                cand = m + (jnp.int32(1) << (16 - j).astype(jnp.int32))
                c = count_eq_lt(cand)
                return jnp.where(c < need, cand, m)

            m = lax.fori_loop(0, 17, m_iter, jnp.zeros((B, 1), jnp.int32))
            m_ref[...] = jnp.broadcast_to(m + 1, (B, 128))

    @pl.when(i >= NB)
    def _decode():
        blk = i - NB
        s = sc_ref[:, pl.ds(pl.multiple_of(blk * BFD, BFD), BFD)]
        tf32 = _unmono(t_ref[:, 0:1])
        m = m_ref[:, 0:1]
        idx = lax.broadcasted_iota(jnp.int32, (B, BFD), 1) + blk * BFD
        sel = (s > tf32) | ((s == tf32) & (idx < m))
        vals = jnp.where(sel, s, 0.0).astype(jnp.bfloat16)
        recon_ref[...] += lax.dot_general(
            vals, wdec_ref[...], (((1,), (0,)), ((), ())),
            preferred_element_type=jnp.float32)

    @pl.when(i == NB + NBD - 1)
    def _finish():
        recon = recon_ref[...] + bias_ref[...]
        s = jnp.sum(recon * recon, axis=1, keepdims=True)
        norm = jnp.maximum(jnp.sqrt(s), 1e-12)
        out_ref[...] = recon / norm


def kernel(embed, W_enc, W_dec, bias):
    bias2 = bias.reshape(1, D)
    wdec_bf16 = W_dec.astype(jnp.bfloat16)
    return pl.pallas_call(
        _body,
        grid=(NB + NBD,),
        in_specs=[
            pl.BlockSpec((B, D), lambda i: (0, 0)),
            pl.BlockSpec((1, D), lambda i: (0, 0)),
            pl.BlockSpec((BF, D), lambda i: (jnp.minimum(i, NB - 1), 0)),
            pl.BlockSpec((BFD, D), lambda i: (jnp.maximum(i - NB, 0), 0)),
        ],
        out_specs=pl.BlockSpec((B, D), lambda i: (0, 0)),
        out_shape=jax.ShapeDtypeStruct((B, D), jnp.float32),
        scratch_shapes=[
            pltpu.VMEM((B, F), jnp.float32),
            pltpu.VMEM((B, D), jnp.float32),
            pltpu.VMEM((B, D), jnp.float32),
            pltpu.VMEM((B, 128), jnp.int32),
            pltpu.VMEM((B, 128), jnp.int32),
            pltpu.VMEM((B, NLVL * NBLK), jnp.float32),
        ],
    )(embed, bias2, W_enc, wdec_bf16)


# speedup vs baseline: 6.5744x; 1.0395x over previous
import jax
import jax.numpy as jnp
from jax import lax
from jax.experimental import pallas as pl
from jax.experimental.pallas import tpu as pltpu

B = 128
D = 768
F = 65536
K = 64
BF = 1024
NB = F // BF
BFD = 4096
NBD = F // BFD
CW = 2048
NCH = F // CW
NBLK = 512
NSTR = F // NBLK
NLVL = 6

_I32_MIN = -2147483648
_MASK31 = 0x7FFFFFFF
_NEG_INF = float("-inf")


def _mono_key(x):
    b = lax.bitcast_convert_type(x, jnp.int32)
    return jnp.where(b < 0, b ^ _MASK31, b)


def _unmono(k):
    b = jnp.where(k < 0, k ^ _MASK31, k)
    return lax.bitcast_convert_type(b, jnp.float32)


def _body(embed_ref, bias_ref, wenc_ref, wdec_ref, out_ref,
          sc_ref, x_ref, recon_ref, t_ref, m_ref, mt_ref):
    i = pl.program_id(0)

    @pl.when(i == 0)
    def _init():
        x_ref[...] = embed_ref[...] - bias_ref[...]
        recon_ref[...] = jnp.zeros((B, D), jnp.float32)
        mt_ref[...] = jnp.full((B, NLVL * NBLK), _NEG_INF, jnp.float32)

    @pl.when(i < NB)
    def _encode():
        s = lax.dot_general(x_ref[...], wenc_ref[...],
                            (((1,), (1,)), ((), ())),
                            preferred_element_type=jnp.float32)
        sc_ref[:, pl.ds(pl.multiple_of(i * BF, BF), BF)] = s
        for sub in range(BF // NBLK):
            v = s[:, sub * NBLK:(sub + 1) * NBLK]
            for l in range(NLVL):
                cur = mt_ref[:, l * NBLK:(l + 1) * NBLK]
                hi = jnp.maximum(cur, v)
                v = jnp.minimum(cur, v)
                mt_ref[:, l * NBLK:(l + 1) * NBLK] = hi

    @pl.when(i == NB - 1)
    def _select():
        def count_ge(cand):
            def chunk(c, acc):
                s = sc_ref[:, pl.ds(pl.multiple_of(c * CW, CW), CW)]
                hit = (_mono_key(s) >= cand).astype(jnp.int32)
                return acc + jnp.sum(hit, axis=1, keepdims=True)
            return lax.fori_loop(0, NCH, chunk, jnp.zeros((B, 1), jnp.int32))

        def count_ge2(cand):
            def chunk(c, accs):
                a1, a2 = accs
                s = sc_ref[:, pl.ds(pl.multiple_of(c * CW, CW), CW)]
                k = _mono_key(s)
                h1 = (k >= cand).astype(jnp.int32)
                h2 = (k >= cand + 1).astype(jnp.int32)
                return (a1 + jnp.sum(h1, axis=1, keepdims=True),
                        a2 + jnp.sum(h2, axis=1, keepdims=True))
            z = jnp.zeros((B, 1), jnp.int32)
            return lax.fori_loop(0, NCH, chunk, (z, z))

        mkeys = _mono_key(lax.transpose(mt_ref[...], (1, 0)))

        def mcount(cand):
            return jnp.sum((mkeys >= cand).astype(jnp.int32),
                           axis=0, keepdims=True)

        def ms_iter(j, t):
            cand = t + (jnp.int32(1) << (30 - j).astype(jnp.int32))
            return jnp.where(mcount(cand) >= K, cand, t)

        tm0 = jnp.where(mcount(jnp.zeros((1, B), jnp.int32)) >= K,
                        0, _I32_MIN).astype(jnp.int32)
        tm = lax.fori_loop(0, 31, ms_iter, tm0)

        def to_col(row):
            return lax.transpose(jnp.broadcast_to(row, (B, B)), (1, 0))[:, 0:1]

        tm_col = to_col(tm)
        mc_col = to_col(mcount(tm))
        cge_tm, cgt_tm = count_ge2(tm_col)
        ok = jnp.sum(jnp.where(cge_tm == mc_col, 0, 1)) == 0
        any_tie = jnp.sum(jnp.where(cge_tm != K, 1, 0)) > 0

        t_ref[...] = jnp.broadcast_to(tm_col, (B, 128))
        m_ref[...] = jnp.full((B, 128), F, jnp.int32)

        @pl.when(jnp.logical_not(ok) | any_tie)
        def _slow_exact():
            def bs_iter(j, t):
                cand = t + (jnp.int32(1) << (30 - j).astype(jnp.int32))
                return jnp.where(count_ge(cand) >= K, cand, t)

            t0 = jnp.where(count_ge(jnp.zeros((B, 1), jnp.int32)) >= K,
                           0, _I32_MIN).astype(jnp.int32)
            tf = lax.fori_loop(0, 31, bs_iter, t0)
            _, cgt_f = count_ge2(tf)
            t_ref[...] = jnp.broadcast_to(tf, (B, 128))
            need = K - cgt_f
            tf32 = _unmono(tf)

            def count_eq_lt(cand):
                def chunk(c, acc):
                    s = sc_ref[:, pl.ds(pl.multiple_of(c * CW, CW), CW)]
                    idx = (lax.broadcasted_iota(jnp.int32, (B, CW), 1)
                           + c * CW)
                    hit = ((s == tf32) & (idx < cand)).astype(jnp.int32)
                    return acc + jnp.sum(hit, axis=1, keepdims=True)
                return lax.fori_loop(0, NCH, chunk,
                                     jnp.zeros((B, 1), jnp.int32))

            def m_iter(j, m):
                cand = m + (jnp.int32(1) << (16 - j).astype(jnp.int32))
                c = count_eq_lt(cand)
                return jnp.where(c < need, cand, m)

            m = lax.fori_loop(0, 17, m_iter, jnp.zeros((B, 1), jnp.int32))
            m_ref[...] = jnp.broadcast_to(m + 1, (B, 128))

    @pl.when(i >= NB)
    def _decode():
        blk = i - NB
        s = sc_ref[:, pl.ds(pl.multiple_of(blk * BFD, BFD), BFD)]
        tf32 = _unmono(t_ref[:, 0:1])
        m = m_ref[:, 0:1]
        idx = lax.broadcasted_iota(jnp.int32, (B, BFD), 1) + blk * BFD
        sel = (s > tf32) | ((s == tf32) & (idx < m))
        vals = jnp.where(sel, s, 0.0).astype(jnp.bfloat16)
        recon_ref[...] += lax.dot_general(
            vals, wdec_ref[...], (((1,), (0,)), ((), ())),
            preferred_element_type=jnp.float32)

    @pl.when(i == NB + NBD - 1)
    def _finish():
        recon = recon_ref[...] + bias_ref[...]
        s = jnp.sum(recon * recon, axis=1, keepdims=True)
        norm = jnp.maximum(jnp.sqrt(s), 1e-12)
        out_ref[...] = recon / norm


def kernel(embed, W_enc, W_dec, bias):
    bias2 = bias.reshape(1, D)
    wdec_bf16 = W_dec.astype(jnp.bfloat16)
    return pl.pallas_call(
        _body,
        grid=(NB + NBD,),
        in_specs=[
            pl.BlockSpec((B, D), lambda i: (0, 0)),
            pl.BlockSpec((1, D), lambda i: (0, 0)),
            pl.BlockSpec((BF, D), lambda i: (jnp.minimum(i, NB - 1), 0)),
            pl.BlockSpec((BFD, D), lambda i: (jnp.maximum(i - NB, 0), 0)),
        ],
        out_specs=pl.BlockSpec((B, D), lambda i: (0, 0)),
        out_shape=jax.ShapeDtypeStruct((B, D), jnp.float32),
        scratch_shapes=[
            pltpu.VMEM((B, F), jnp.float32),
            pltpu.VMEM((B, D), jnp.float32),
            pltpu.VMEM((B, D), jnp.float32),
            pltpu.VMEM((B, 128), jnp.int32),
            pltpu.VMEM((B, 128), jnp.int32),
            pltpu.VMEM((B, NLVL * NBLK), jnp.float32),
        ],
    )(embed, bias2, W_enc, wdec_bf16)
